# baseline (speedup 1.0000x reference)
import functools

import jax
import jax.numpy as jnp
from jax import lax
from jax.experimental import pallas as pl
from jax.experimental.pallas import tpu as pltpu
from jax.experimental.pallas import tpu_sc as plsc

L = 2; D = 768; H = 12; DH = 64; FF = 1536; E = 8; CW = 2048; S = 2048
HALF = DH // 2

TS = 256
TQ = 512
TM = 128
NF = 3
FB = FF // NF
P = S + E * TM
NT = P // TM

_f32 = jnp.float32


def _qkv_body(x_ref, ln_ref, wq_ref, wk_ref, wv_ref, q_ref, k_ref, v_ref):
    x = x_ref[...]
    h = x * lax.rsqrt(jnp.mean(x * x, axis=-1, keepdims=True) + 1e-5) * ln_ref[...]
    q_ref[...] = jnp.dot(h, wq_ref[...], preferred_element_type=_f32)
    k_ref[...] = jnp.dot(h, wk_ref[...], preferred_element_type=_f32)
    v_ref[...] = jnp.dot(h, wv_ref[...], preferred_element_type=_f32)


def _qkv(x, ln_w, wq, wk, wv):
    return pl.pallas_call(
        _qkv_body,
        grid=(S // TS,),
        in_specs=[
            pl.BlockSpec((TS, D), lambda i: (i, 0)),
            pl.BlockSpec((1, D), lambda i: (0, 0)),
            pl.BlockSpec((D, D), lambda i: (0, 0)),
            pl.BlockSpec((D, D), lambda i: (0, 0)),
            pl.BlockSpec((D, D), lambda i: (0, 0)),
        ],
        out_specs=[
            pl.BlockSpec((TS, D), lambda i: (i, 0)),
            pl.BlockSpec((TS, D), lambda i: (i, 0)),
            pl.BlockSpec((TS, D), lambda i: (i, 0)),
        ],
        out_shape=[jax.ShapeDtypeStruct((S, D), _f32)] * 3,
    )(x, ln_w.reshape(1, D), wq, wk, wv)


def _att_body(q_ref, k_ref, v_ref, cos_ref, sin_ref, o_ref):
    i = pl.program_id(1)
    q = q_ref[0]
    k = k_ref[0]
    cq = cos_ref[pl.ds(i * TQ, TQ), :]
    sq = sin_ref[pl.ds(i * TQ, TQ), :]
    q1, q2 = q[:, :HALF], q[:, HALF:]
    qr = jnp.concatenate([q1 * cq - q2 * sq, q1 * sq + q2 * cq], axis=-1)
    ck = cos_ref[...]
    sk = sin_ref[...]
    k1, k2 = k[:, :HALF], k[:, HALF:]
    kr = jnp.concatenate([k1 * ck - k2 * sk, k1 * sk + k2 * ck], axis=-1)
    s = lax.dot_general(qr, kr, (((1,), (1,)), ((), ())),
                        preferred_element_type=_f32) * (1.0 / 8.0)
    row = i * TQ + lax.broadcasted_iota(jnp.int32, (TQ, S), 0)
    col = lax.broadcasted_iota(jnp.int32, (TQ, S), 1)
    s = jnp.where(col <= row, s, _f32(-1e30))
    m = jnp.max(s, axis=-1, keepdims=True)
    p = jnp.exp(s - m)
    o = jnp.dot(p, v_ref[0], preferred_element_type=_f32)
    o_ref[0] = o / jnp.sum(p, axis=-1, keepdims=True)


def _attention(q, k, v, cos, sin):
    o = pl.pallas_call(
        _att_body,
        grid=(H, S // TQ),
        in_specs=[
            pl.BlockSpec((1, TQ, DH), lambda h, i: (h, i, 0)),
            pl.BlockSpec((1, S, DH), lambda h, i: (h, 0, 0)),
            pl.BlockSpec((1, S, DH), lambda h, i: (h, 0, 0)),
            pl.BlockSpec((S, HALF), lambda h, i: (0, 0)),
            pl.BlockSpec((S, HALF), lambda h, i: (0, 0)),
        ],
        out_specs=pl.BlockSpec((1, TQ, DH), lambda h, i: (h, i, 0)),
        out_shape=jax.ShapeDtypeStruct((H, S, DH), _f32),
    )(q, k, v, cos, sin)
    return o


def _proj_body(o_ref, w_ref, x_ref, out_ref):
    out_ref[...] = x_ref[...] + jnp.dot(o_ref[...], w_ref[...],
                                        preferred_element_type=_f32)


def _proj_residual(o, wo, x):
    return pl.pallas_call(
        _proj_body,
        grid=(S // TS,),
        in_specs=[
            pl.BlockSpec((TS, D), lambda i: (i, 0)),
            pl.BlockSpec((D, D), lambda i: (0, 0)),
            pl.BlockSpec((TS, D), lambda i: (i, 0)),
        ],
        out_specs=pl.BlockSpec((TS, D), lambda i: (i, 0)),
        out_shape=jax.ShapeDtypeStruct((S, D), _f32),
    )(o, wo, x)


def _router_body(x_ref, ln_ref, rw_ref, h_ref, probs_ref, fsum_ref, psum_ref):
    i = pl.program_id(0)
    x = x_ref[...]
    h = x * lax.rsqrt(jnp.mean(x * x, axis=-1, keepdims=True) + 1e-5) * ln_ref[...]
    h_ref[...] = h
    logits = jnp.dot(h, rw_ref[...], preferred_element_type=_f32)
    m = jnp.max(logits, axis=-1, keepdims=True)
    ex = jnp.exp(logits - m)
    probs = ex / jnp.sum(ex, axis=-1, keepdims=True)
    probs_ref[...] = probs
    mp = jnp.max(probs, axis=-1, keepdims=True)
    ie = lax.broadcasted_iota(jnp.int32, (TS, E), 1)
    sel = jnp.min(jnp.where(probs == mp, ie, E), axis=-1, keepdims=True)
    onehot = (ie == sel).astype(_f32)

    @pl.when(i == 0)
    def _():
        fsum_ref[...] = jnp.zeros_like(fsum_ref)
        psum_ref[...] = jnp.zeros_like(psum_ref)

    fsum_ref[...] += jnp.sum(onehot, axis=0, keepdims=True)
    psum_ref[...] += jnp.sum(probs, axis=0, keepdims=True)


def _router(x, ln_w, rw):
    return pl.pallas_call(
        _router_body,
        grid=(S // TS,),
        in_specs=[
            pl.BlockSpec((TS, D), lambda i: (i, 0)),
            pl.BlockSpec((1, D), lambda i: (0, 0)),
            pl.BlockSpec((D, E), lambda i: (0, 0)),
        ],
        out_specs=[
            pl.BlockSpec((TS, D), lambda i: (i, 0)),
            pl.BlockSpec((TS, E), lambda i: (i, 0)),
            pl.BlockSpec((1, E), lambda i: (0, 0)),
            pl.BlockSpec((1, E), lambda i: (0, 0)),
        ],
        out_shape=[
            jax.ShapeDtypeStruct((S, D), _f32),
            jax.ShapeDtypeStruct((S, E), _f32),
            jax.ShapeDtypeStruct((1, E), _f32),
            jax.ShapeDtypeStruct((1, E), _f32),
        ],
    )(x, ln_w.reshape(1, D), rw)


def _moe_body(te_ref, na_ref, x_ref, w1_ref, w3_ref, w2_ref, ws_ref,
              out_ref, acc_ref):
    i = pl.program_id(0)
    c = pl.program_id(1)

    @pl.when(c == 0)
    def _():
        acc_ref[...] = jnp.zeros_like(acc_ref)

    @pl.when(i * TM < na_ref[0])
    def _():
        x = x_ref[...]
        h1 = jnp.dot(x, w1_ref[0], preferred_element_type=_f32)
        h3 = jnp.dot(x, w3_ref[0], preferred_element_type=_f32)
        g = jax.nn.silu(h1) * h3
        acc_ref[...] += jnp.dot(g, w2_ref[0], preferred_element_type=_f32)

    @pl.when(c == NF - 1)
    def _():
        out_ref[...] = acc_ref[...] * ws_ref[...]


def _moe(xp, w1, w3, w2, wslot, tile_e, n_active):
    grid_spec = pltpu.PrefetchScalarGridSpec(
        num_scalar_prefetch=2,
        grid=(NT, NF),
        in_specs=[
            pl.BlockSpec((TM, D), lambda i, c, te, na: (i, 0)),
            pl.BlockSpec((1, D, FB), lambda i, c, te, na: (te[i], 0, c)),
            pl.BlockSpec((1, D, FB), lambda i, c, te, na: (te[i], 0, c)),
            pl.BlockSpec((1, FB, D), lambda i, c, te, na: (te[i], c, 0)),
            pl.BlockSpec((TM, 1), lambda i, c, te, na: (i, 0)),
        ],
        out_specs=pl.BlockSpec((TM, D), lambda i, c, te, na: (i, 0)),
        scratch_shapes=[pltpu.VMEM((TM, D), _f32)],
    )
    return pl.pallas_call(
        _moe_body,
        grid_spec=grid_spec,
        out_shape=jax.ShapeDtypeStruct((P, D), _f32),
    )(tile_e, n_active, xp, w1, w3, w2, wslot)


_NC = 2
_NS = 16
_NW = _NC * _NS
_BPW = P // _NW


def _sc_gather_body(table_hbm, idx_hbm, out_hbm, idx_v, rows_v, sem):
    wid = lax.axis_index("s") * _NC + lax.axis_index("c")
    base = wid * _BPW
    pltpu.sync_copy(idx_hbm.at[pl.ds(base, _BPW)], idx_v)
    pltpu.async_copy(table_hbm.at[idx_v], rows_v, sem).wait()
    pltpu.sync_copy(rows_v, out_hbm.at[pl.ds(base, _BPW)])


def _gather(table, idx):
    f = functools.partial(
        pl.kernel,
        mesh=plsc.VectorSubcoreMesh(core_axis_name="c", subcore_axis_name="s"),
        out_type=jax.ShapeDtypeStruct((P, D), _f32),
        scratch_types=[
            pltpu.VMEM((_BPW,), jnp.int32),
            pltpu.VMEM((_BPW, D), _f32),
            pltpu.SemaphoreType.DMA,
        ],
    )(_sc_gather_body)
    return f(table, idx)


def _sc_scatter_body(y_hbm, idx_hbm, out_hbm, idx_v, rows_v, sem):
    wid = lax.axis_index("s") * _NC + lax.axis_index("c")
    base = wid * _BPW
    pltpu.sync_copy(idx_hbm.at[pl.ds(base, _BPW)], idx_v)
    pltpu.sync_copy(y_hbm.at[pl.ds(base, _BPW)], rows_v)
    pltpu.async_copy(rows_v, out_hbm.at[idx_v], sem).wait()


def _scatter(y, idx):
    f = functools.partial(
        pl.kernel,
        mesh=plsc.VectorSubcoreMesh(core_axis_name="c", subcore_axis_name="s"),
        out_type=jax.ShapeDtypeStruct((S + 8, D), _f32),
        scratch_types=[
            pltpu.VMEM((_BPW,), jnp.int32),
            pltpu.VMEM((_BPW, D), _f32),
            pltpu.SemaphoreType.DMA,
        ],
    )(_sc_scatter_body)
    return f(y, idx)[:S]


def _route_schedule(probs):
    eid = jnp.argmax(probs, axis=-1).astype(jnp.int32)
    topv = jnp.max(probs, axis=-1)
    counts = jnp.zeros(E, jnp.int32).at[eid].add(1)
    start = jnp.concatenate([jnp.zeros(1, jnp.int32),
                             jnp.cumsum(counts)[:-1]])
    pc = ((counts + TM - 1) // TM) * TM
    pend = jnp.cumsum(pc)
    poff = jnp.concatenate([jnp.zeros(1, jnp.int32), pend[:-1]])
    order = jnp.argsort(eid, stable=True).astype(jnp.int32)
    eids = eid[order]
    slots = poff[eids] + (jnp.arange(S, dtype=jnp.int32) - start[eids])
    gidx = jnp.zeros(P, jnp.int32).at[slots].set(order)
    sidx = jnp.full(P, S, jnp.int32).at[slots].set(order)
    wslot = jnp.zeros((P, 1), _f32).at[slots, 0].set(topv[order])
    tile_e = jnp.minimum(
        jnp.searchsorted(pend, jnp.arange(NT, dtype=jnp.int32) * TM,
                         side="right"),
        E - 1).astype(jnp.int32)
    n_active = pend[-1:].astype(jnp.int32)
    return eid, topv, gidx, sidx, wslot, tile_e, n_active


def kernel(x, pos_emb, ln1_w, ln2_w, wq, wk, wv, wo, router_w, w1, w2, w3):
    xs = x.reshape(S, D) + pos_emb[:S]

    inv = 1.0 / (10000.0 ** (jnp.arange(HALF, dtype=_f32) / HALF))
    ang = jnp.arange(S, dtype=_f32)[:, None] * inv[None, :]
    cos = jnp.cos(ang)
    sin = jnp.sin(ang)

    total_aux = jnp.zeros((), _f32)
    for l in range(L):
        q, k, v = _qkv(xs, ln1_w[l], wq[l], wk[l], wv[l])
        q3 = q.reshape(S, H, DH).transpose(1, 0, 2)
        k3 = k.reshape(S, H, DH).transpose(1, 0, 2)
        v3 = v.reshape(S, H, DH).transpose(1, 0, 2)
        o = _attention(q3, k3, v3, cos, sin)
        o2 = o.transpose(1, 0, 2).reshape(S, D)
        xs = _proj_residual(o2, wo[l], xs)

        h2, probs, fsum, psum = _router(xs, ln2_w[l], router_w[l])
        _, _, gidx, sidx, wslot, tile_e, n_active = _route_schedule(probs)
        xp = _gather(h2, gidx)
        yp = _moe(xp, w1[l], w3[l], w2[l], wslot, tile_e, n_active)
        moe_out = _scatter(yp, sidx)
        xs = xs + moe_out

        total_aux = total_aux + _f32(E) * jnp.sum(
            (fsum[0] / _f32(S)) * (psum[0] / _f32(S)))

    return xs.reshape(1, S, D), total_aux


# speedup vs baseline: 1.2356x; 1.2356x over previous
import jax
import jax.numpy as jnp
from jax import lax
from jax.experimental import pallas as pl
from jax.experimental.pallas import tpu as pltpu

L = 2; D = 768; H = 12; DH = 64; FF = 1536; E = 8; CW = 2048; S = 2048
HALF = DH // 2

TS = 256
TQ = 512
TM = 128
NF = 3
FB = FF // NF
P = S + E * TM
NT = P // TM

_f32 = jnp.float32


def _qkv_body(x_ref, ln_ref, wq_ref, wk_ref, wv_ref, q_ref, k_ref, v_ref):
    x = x_ref[...]
    h = x * lax.rsqrt(jnp.mean(x * x, axis=-1, keepdims=True) + 1e-5) * ln_ref[...]
    q_ref[...] = jnp.dot(h, wq_ref[...], preferred_element_type=_f32)
    k_ref[...] = jnp.dot(h, wk_ref[...], preferred_element_type=_f32)
    v_ref[...] = jnp.dot(h, wv_ref[...], preferred_element_type=_f32)


def _qkv(x, ln_w, wq, wk, wv):
    return pl.pallas_call(
        _qkv_body,
        grid=(S // TS,),
        in_specs=[
            pl.BlockSpec((TS, D), lambda i: (i, 0)),
            pl.BlockSpec((1, D), lambda i: (0, 0)),
            pl.BlockSpec((D, D), lambda i: (0, 0)),
            pl.BlockSpec((D, D), lambda i: (0, 0)),
            pl.BlockSpec((D, D), lambda i: (0, 0)),
        ],
        out_specs=[
            pl.BlockSpec((TS, D), lambda i: (i, 0)),
            pl.BlockSpec((TS, D), lambda i: (i, 0)),
            pl.BlockSpec((TS, D), lambda i: (i, 0)),
        ],
        out_shape=[jax.ShapeDtypeStruct((S, D), _f32)] * 3,
    )(x, ln_w.reshape(1, D), wq, wk, wv)


def _att_body(q_ref, k_ref, v_ref, cos_ref, sin_ref, o_ref):
    i = pl.program_id(1)
    q = q_ref[0]
    k = k_ref[0]
    cq = cos_ref[pl.ds(i * TQ, TQ), :]
    sq = sin_ref[pl.ds(i * TQ, TQ), :]
    q1, q2 = q[:, :HALF], q[:, HALF:]
    qr = jnp.concatenate([q1 * cq - q2 * sq, q1 * sq + q2 * cq], axis=-1)
    ck = cos_ref[...]
    sk = sin_ref[...]
    k1, k2 = k[:, :HALF], k[:, HALF:]
    kr = jnp.concatenate([k1 * ck - k2 * sk, k1 * sk + k2 * ck], axis=-1)
    s = lax.dot_general(qr, kr, (((1,), (1,)), ((), ())),
                        preferred_element_type=_f32) * (1.0 / 8.0)
    row = i * TQ + lax.broadcasted_iota(jnp.int32, (TQ, S), 0)
    col = lax.broadcasted_iota(jnp.int32, (TQ, S), 1)
    s = jnp.where(col <= row, s, _f32(-1e30))
    m = jnp.max(s, axis=-1, keepdims=True)
    p = jnp.exp(s - m)
    o = jnp.dot(p, v_ref[0], preferred_element_type=_f32)
    o_ref[0] = o / jnp.sum(p, axis=-1, keepdims=True)


def _attention(q, k, v, cos, sin):
    return pl.pallas_call(
        _att_body,
        grid=(H, S // TQ),
        in_specs=[
            pl.BlockSpec((1, TQ, DH), lambda h, i: (h, i, 0)),
            pl.BlockSpec((1, S, DH), lambda h, i: (h, 0, 0)),
            pl.BlockSpec((1, S, DH), lambda h, i: (h, 0, 0)),
            pl.BlockSpec((S, HALF), lambda h, i: (0, 0)),
            pl.BlockSpec((S, HALF), lambda h, i: (0, 0)),
        ],
        out_specs=pl.BlockSpec((1, TQ, DH), lambda h, i: (h, i, 0)),
        out_shape=jax.ShapeDtypeStruct((H, S, DH), _f32),
    )(q, k, v, cos, sin)


def _proj_body(o_ref, w_ref, x_ref, out_ref):
    out_ref[...] = x_ref[...] + jnp.dot(o_ref[...], w_ref[...],
                                        preferred_element_type=_f32)


def _proj_residual(o, wo, x):
    return pl.pallas_call(
        _proj_body,
        grid=(S // TS,),
        in_specs=[
            pl.BlockSpec((TS, D), lambda i: (i, 0)),
            pl.BlockSpec((D, D), lambda i: (0, 0)),
            pl.BlockSpec((TS, D), lambda i: (i, 0)),
        ],
        out_specs=pl.BlockSpec((TS, D), lambda i: (i, 0)),
        out_shape=jax.ShapeDtypeStruct((S, D), _f32),
    )(o, wo, x)


def _router_body(x_ref, ln_ref, rw_ref, h_ref, probs_ref, rank_ref,
                 fsum_ref, psum_ref):
    i = pl.program_id(0)
    x = x_ref[...]
    h = x * lax.rsqrt(jnp.mean(x * x, axis=-1, keepdims=True) + 1e-5) * ln_ref[...]
    h_ref[...] = h
    logits = jnp.dot(h, rw_ref[...], preferred_element_type=_f32)
    m = jnp.max(logits, axis=-1, keepdims=True)
    ex = jnp.exp(logits - m)
    probs = ex / jnp.sum(ex, axis=-1, keepdims=True)
    probs_ref[...] = probs
    mp = jnp.max(probs, axis=-1, keepdims=True)
    ie = lax.broadcasted_iota(jnp.int32, (TS, E), 1)
    sel = jnp.min(jnp.where(probs == mp, ie, E), axis=-1, keepdims=True)
    onehot = (ie == sel).astype(_f32)

    @pl.when(i == 0)
    def _():
        fsum_ref[...] = jnp.zeros_like(fsum_ref)
        psum_ref[...] = jnp.zeros_like(psum_ref)

    r0 = lax.broadcasted_iota(jnp.int32, (TS, TS), 0)
    c0 = lax.broadcasted_iota(jnp.int32, (TS, TS), 1)
    lt = (c0 < r0).astype(_f32)
    local = jnp.dot(lt, onehot, preferred_element_type=_f32)
    rank_ref[...] = jnp.sum(onehot * (fsum_ref[...] + local), axis=-1,
                            keepdims=True)

    fsum_ref[...] += jnp.sum(onehot, axis=0, keepdims=True)
    psum_ref[...] += jnp.sum(probs, axis=0, keepdims=True)


def _router(x, ln_w, rw):
    return pl.pallas_call(
        _router_body,
        grid=(S // TS,),
        in_specs=[
            pl.BlockSpec((TS, D), lambda i: (i, 0)),
            pl.BlockSpec((1, D), lambda i: (0, 0)),
            pl.BlockSpec((D, E), lambda i: (0, 0)),
        ],
        out_specs=[
            pl.BlockSpec((TS, D), lambda i: (i, 0)),
            pl.BlockSpec((TS, E), lambda i: (i, 0)),
            pl.BlockSpec((TS, 1), lambda i: (i, 0)),
            pl.BlockSpec((1, E), lambda i: (0, 0)),
            pl.BlockSpec((1, E), lambda i: (0, 0)),
        ],
        out_shape=[
            jax.ShapeDtypeStruct((S, D), _f32),
            jax.ShapeDtypeStruct((S, E), _f32),
            jax.ShapeDtypeStruct((S, 1), _f32),
            jax.ShapeDtypeStruct((1, E), _f32),
            jax.ShapeDtypeStruct((1, E), _f32),
        ],
    )(x, ln_w.reshape(1, D), rw)


def _moe_body(te_ref, na_ref, slots_ref, h2_ref, w1_ref, w3_ref, w2_ref,
              out_ref, xp_ref, acc_ref):
    i = pl.program_id(0)
    c = pl.program_id(1)
    active = i * TM < na_ref[0]

    @pl.when(jnp.logical_and(c == 0, active))
    def _():
        rows = i * TM + lax.broadcasted_iota(jnp.int32, (TM, S), 0)
        oh = (slots_ref[...] == rows).astype(_f32)
        xp_ref[...] = jnp.dot(oh, h2_ref[...], preferred_element_type=_f32)
        acc_ref[...] = jnp.zeros_like(acc_ref)

    @pl.when(active)
    def _():
        x = xp_ref[...]
        h1 = jnp.dot(x, w1_ref[0], preferred_element_type=_f32)
        h3 = jnp.dot(x, w3_ref[0], preferred_element_type=_f32)
        g = jax.nn.silu(h1) * h3
        acc_ref[...] += jnp.dot(g, w2_ref[0], preferred_element_type=_f32)

    @pl.when(c == NF - 1)
    def _():
        out_ref[...] = acc_ref[...]


def _moe(h2, slots_row, w1, w3, w2, tile_e, n_active):
    grid_spec = pltpu.PrefetchScalarGridSpec(
        num_scalar_prefetch=2,
        grid=(NT, NF),
        in_specs=[
            pl.BlockSpec((1, S), lambda i, c, te, na: (0, 0)),
            pl.BlockSpec((S, D), lambda i, c, te, na: (0, 0)),
            pl.BlockSpec((1, D, FB), lambda i, c, te, na: (te[i], 0, c)),
            pl.BlockSpec((1, D, FB), lambda i, c, te, na: (te[i], 0, c)),
            pl.BlockSpec((1, FB, D), lambda i, c, te, na: (te[i], c, 0)),
        ],
        out_specs=pl.BlockSpec((TM, D), lambda i, c, te, na: (i, 0)),
        scratch_shapes=[pltpu.VMEM((TM, D), _f32), pltpu.VMEM((TM, D), _f32)],
    )
    return pl.pallas_call(
        _moe_body,
        grid_spec=grid_spec,
        out_shape=jax.ShapeDtypeStruct((P, D), _f32),
    )(tile_e, n_active, slots_row, h2, w1, w3, w2)


def _combine_body(slots_ref, topv_ref, yp_ref, x_ref, out_ref):
    cols = lax.broadcasted_iota(jnp.int32, (TS, P), 1)
    oh = (slots_ref[...] == cols).astype(_f32)
    y = jnp.dot(oh, yp_ref[...], preferred_element_type=_f32)
    out_ref[...] = x_ref[...] + topv_ref[...] * y


def _combine(slots_col, topv, yp, x):
    return pl.pallas_call(
        _combine_body,
        grid=(S // TS,),
        in_specs=[
            pl.BlockSpec((TS, 1), lambda i: (i, 0)),
            pl.BlockSpec((TS, 1), lambda i: (i, 0)),
            pl.BlockSpec((P, D), lambda i: (0, 0)),
            pl.BlockSpec((TS, D), lambda i: (i, 0)),
        ],
        out_specs=pl.BlockSpec((TS, D), lambda i: (i, 0)),
        out_shape=jax.ShapeDtypeStruct((S, D), _f32),
    )(slots_col, topv, yp, x)


def _route_schedule(probs, fsum, rank):
    eid = jnp.argmax(probs, axis=-1).astype(jnp.int32)
    topv = jnp.max(probs, axis=-1, keepdims=True)
    counts = fsum[0].astype(jnp.int32)
    pc = ((counts + TM - 1) // TM) * TM
    pend = jnp.cumsum(pc)
    poff = jnp.concatenate([jnp.zeros(1, jnp.int32), pend[:-1]])
    slots = jnp.take(poff, eid) + rank[:, 0].astype(jnp.int32)
    tile_e = jnp.minimum(
        jnp.searchsorted(pend, jnp.arange(NT, dtype=jnp.int32) * TM,
                         side="right"),
        E - 1).astype(jnp.int32)
    n_active = pend[-1:].astype(jnp.int32)
    return slots, topv, tile_e, n_active


def kernel(x, pos_emb, ln1_w, ln2_w, wq, wk, wv, wo, router_w, w1, w2, w3):
    xs = x.reshape(S, D) + pos_emb[:S]

    inv = 1.0 / (10000.0 ** (jnp.arange(HALF, dtype=_f32) / HALF))
    ang = jnp.arange(S, dtype=_f32)[:, None] * inv[None, :]
    cos = jnp.cos(ang)
    sin = jnp.sin(ang)

    total_aux = jnp.zeros((), _f32)
    for l in range(L):
        q, k, v = _qkv(xs, ln1_w[l], wq[l], wk[l], wv[l])
        q3 = q.reshape(S, H, DH).transpose(1, 0, 2)
        k3 = k.reshape(S, H, DH).transpose(1, 0, 2)
        v3 = v.reshape(S, H, DH).transpose(1, 0, 2)
        o = _attention(q3, k3, v3, cos, sin)
        o2 = o.transpose(1, 0, 2).reshape(S, D)
        xs = _proj_residual(o2, wo[l], xs)

        h2, probs, rank, fsum, psum = _router(xs, ln2_w[l], router_w[l])
        slots, topv, tile_e, n_active = _route_schedule(probs, fsum, rank)
        yp = _moe(h2, slots.reshape(1, S), w1[l], w3[l], w2[l],
                  tile_e, n_active)
        xs = _combine(slots.reshape(S, 1), topv, yp, xs)

        total_aux = total_aux + _f32(E) * jnp.sum(
            (fsum[0] / _f32(S)) * (psum[0] / _f32(S)))

    return xs.reshape(1, S, D), total_aux


# speedup vs baseline: 1.2777x; 1.0341x over previous
import jax
import jax.numpy as jnp
from jax import lax
from jax.experimental import pallas as pl
from jax.experimental.pallas import tpu as pltpu

L = 2; D = 768; H = 12; DH = 64; FF = 1536; E = 8; CW = 2048; S = 2048
HALF = DH // 2

TS = 256
TQ = 512
TM = 128
NF = 3
FB = FF // NF
P = S + E * TM
NT = P // TM

_f32 = jnp.float32


def _qkv_body(x_ref, ln_ref, wq_ref, wk_ref, wv_ref, q_ref, k_ref, v_ref):
    x = x_ref[...]
    h = x * lax.rsqrt(jnp.mean(x * x, axis=-1, keepdims=True) + 1e-5) * ln_ref[...]
    q_ref[...] = jnp.dot(h, wq_ref[...], preferred_element_type=_f32)
    k_ref[...] = jnp.dot(h, wk_ref[...], preferred_element_type=_f32)
    v_ref[...] = jnp.dot(h, wv_ref[...], preferred_element_type=_f32)


def _qkv(x, ln_w, wq, wk, wv):
    return pl.pallas_call(
        _qkv_body,
        grid=(S // TS,),
        in_specs=[
            pl.BlockSpec((TS, D), lambda i: (i, 0)),
            pl.BlockSpec((1, D), lambda i: (0, 0)),
            pl.BlockSpec((D, D), lambda i: (0, 0)),
            pl.BlockSpec((D, D), lambda i: (0, 0)),
            pl.BlockSpec((D, D), lambda i: (0, 0)),
        ],
        out_specs=[
            pl.BlockSpec((TS, D), lambda i: (i, 0)),
            pl.BlockSpec((TS, D), lambda i: (i, 0)),
            pl.BlockSpec((TS, D), lambda i: (i, 0)),
        ],
        out_shape=[jax.ShapeDtypeStruct((S, D), _f32)] * 3,
    )(x, ln_w.reshape(1, D), wq, wk, wv)


def _att_body(q_ref, k_ref, v_ref, cos_ref, sin_ref, o_ref):
    i = pl.program_id(1)
    q = q_ref[0]
    k = k_ref[0]
    cq = cos_ref[pl.ds(i * TQ, TQ), :]
    sq = sin_ref[pl.ds(i * TQ, TQ), :]
    q1, q2 = q[:, :HALF], q[:, HALF:]
    qr = jnp.concatenate([q1 * cq - q2 * sq, q1 * sq + q2 * cq], axis=-1)
    ck = cos_ref[...]
    sk = sin_ref[...]
    k1, k2 = k[:, :HALF], k[:, HALF:]
    kr = jnp.concatenate([k1 * ck - k2 * sk, k1 * sk + k2 * ck], axis=-1)
    s = lax.dot_general(qr, kr, (((1,), (1,)), ((), ())),
                        preferred_element_type=_f32) * (1.0 / 8.0)
    row = i * TQ + lax.broadcasted_iota(jnp.int32, (TQ, S), 0)
    col = lax.broadcasted_iota(jnp.int32, (TQ, S), 1)
    s = jnp.where(col <= row, s, _f32(-1e30))
    m = jnp.max(s, axis=-1, keepdims=True)
    p = jnp.exp(s - m)
    o = jnp.dot(p, v_ref[0], preferred_element_type=_f32)
    o_ref[0] = o / jnp.sum(p, axis=-1, keepdims=True)


def _attention(q, k, v, cos, sin):
    return pl.pallas_call(
        _att_body,
        grid=(H, S // TQ),
        in_specs=[
            pl.BlockSpec((1, TQ, DH), lambda h, i: (h, i, 0)),
            pl.BlockSpec((1, S, DH), lambda h, i: (h, 0, 0)),
            pl.BlockSpec((1, S, DH), lambda h, i: (h, 0, 0)),
            pl.BlockSpec((S, HALF), lambda h, i: (0, 0)),
            pl.BlockSpec((S, HALF), lambda h, i: (0, 0)),
        ],
        out_specs=pl.BlockSpec((1, TQ, DH), lambda h, i: (h, i, 0)),
        out_shape=jax.ShapeDtypeStruct((H, S, DH), _f32),
    )(q, k, v, cos, sin)


def _proj_body(o_ref, w_ref, x_ref, out_ref):
    out_ref[...] = x_ref[...] + jnp.dot(o_ref[...], w_ref[...],
                                        preferred_element_type=_f32)


def _proj_residual(o, wo, x):
    return pl.pallas_call(
        _proj_body,
        grid=(S // TS,),
        in_specs=[
            pl.BlockSpec((TS, D), lambda i: (i, 0)),
            pl.BlockSpec((D, D), lambda i: (0, 0)),
            pl.BlockSpec((TS, D), lambda i: (i, 0)),
        ],
        out_specs=pl.BlockSpec((TS, D), lambda i: (i, 0)),
        out_shape=jax.ShapeDtypeStruct((S, D), _f32),
    )(o, wo, x)


def _router_body(x_ref, ln_ref, rw_ref, h_ref, probs_ref, rank_ref,
                 fsum_ref, psum_ref):
    i = pl.program_id(0)
    x = x_ref[...]
    h = x * lax.rsqrt(jnp.mean(x * x, axis=-1, keepdims=True) + 1e-5) * ln_ref[...]
    h_ref[...] = h
    logits = jnp.dot(h, rw_ref[...], preferred_element_type=_f32)
    m = jnp.max(logits, axis=-1, keepdims=True)
    ex = jnp.exp(logits - m)
    probs = ex / jnp.sum(ex, axis=-1, keepdims=True)
    probs_ref[...] = probs
    mp = jnp.max(probs, axis=-1, keepdims=True)
    ie = lax.broadcasted_iota(jnp.int32, (TS, E), 1)
    sel = jnp.min(jnp.where(probs == mp, ie, E), axis=-1, keepdims=True)
    onehot = (ie == sel).astype(_f32)

    @pl.when(i == 0)
    def _():
        fsum_ref[...] = jnp.zeros_like(fsum_ref)
        psum_ref[...] = jnp.zeros_like(psum_ref)

    r0 = lax.broadcasted_iota(jnp.int32, (TS, TS), 0)
    c0 = lax.broadcasted_iota(jnp.int32, (TS, TS), 1)
    lt = (c0 < r0).astype(_f32)
    local = jnp.dot(lt, onehot, preferred_element_type=_f32)
    rank_ref[...] = jnp.sum(onehot * (fsum_ref[...] + local), axis=-1,
                            keepdims=True)

    fsum_ref[...] += jnp.sum(onehot, axis=0, keepdims=True)
    psum_ref[...] += jnp.sum(probs, axis=0, keepdims=True)


def _router(x, ln_w, rw):
    return pl.pallas_call(
        _router_body,
        grid=(S // TS,),
        in_specs=[
            pl.BlockSpec((TS, D), lambda i: (i, 0)),
            pl.BlockSpec((1, D), lambda i: (0, 0)),
            pl.BlockSpec((D, E), lambda i: (0, 0)),
        ],
        out_specs=[
            pl.BlockSpec((TS, D), lambda i: (i, 0)),
            pl.BlockSpec((TS, E), lambda i: (i, 0)),
            pl.BlockSpec((TS, 1), lambda i: (i, 0)),
            pl.BlockSpec((1, E), lambda i: (0, 0)),
            pl.BlockSpec((1, E), lambda i: (0, 0)),
        ],
        out_shape=[
            jax.ShapeDtypeStruct((S, D), _f32),
            jax.ShapeDtypeStruct((S, E), _f32),
            jax.ShapeDtypeStruct((S, 1), _f32),
            jax.ShapeDtypeStruct((1, E), _f32),
            jax.ShapeDtypeStruct((1, E), _f32),
        ],
    )(x, ln_w.reshape(1, D), rw)


def _moe_body(te_ref, na_ref, slots_ref, h2_ref, w1_ref, w3_ref, w2_ref,
              out_ref, xp_ref, acc_ref):
    c = pl.program_id(0)
    i = pl.program_id(1)
    active = i * TM < na_ref[0]
    sl = pl.ds(i * TM, TM)

    @pl.when(c == 0)
    def _():
        acc_ref[sl, :] = jnp.zeros((TM, D), _f32)

    @pl.when(jnp.logical_and(c == 0, active))
    def _():
        rows = i * TM + lax.broadcasted_iota(jnp.int32, (TM, S), 0)
        oh = (slots_ref[...] == rows).astype(_f32)
        xp_ref[sl, :] = jnp.dot(oh, h2_ref[...], preferred_element_type=_f32)

    @pl.when(active)
    def _():
        x = xp_ref[sl, :]
        h1 = jnp.dot(x, w1_ref[0], preferred_element_type=_f32)
        h3 = jnp.dot(x, w3_ref[0], preferred_element_type=_f32)
        g = jax.nn.silu(h1) * h3
        acc_ref[sl, :] += jnp.dot(g, w2_ref[0], preferred_element_type=_f32)

    @pl.when(c == NF - 1)
    def _():
        out_ref[...] = acc_ref[sl, :]


def _moe(h2, slots_row, w1, w3, w2, tile_e, n_active):
    grid_spec = pltpu.PrefetchScalarGridSpec(
        num_scalar_prefetch=2,
        grid=(NF, NT),
        in_specs=[
            pl.BlockSpec((1, S), lambda c, i, te, na: (0, 0)),
            pl.BlockSpec((S, D), lambda c, i, te, na: (0, 0)),
            pl.BlockSpec((1, D, FB), lambda c, i, te, na: (te[i], 0, c)),
            pl.BlockSpec((1, D, FB), lambda c, i, te, na: (te[i], 0, c)),
            pl.BlockSpec((1, FB, D), lambda c, i, te, na: (te[i], c, 0)),
        ],
        out_specs=pl.BlockSpec((TM, D), lambda c, i, te, na: (i, 0)),
        scratch_shapes=[pltpu.VMEM((P, D), _f32), pltpu.VMEM((P, D), _f32)],
    )
    return pl.pallas_call(
        _moe_body,
        grid_spec=grid_spec,
        out_shape=jax.ShapeDtypeStruct((P, D), _f32),
    )(tile_e, n_active, slots_row, h2, w1, w3, w2)


def _combine_body(slots_ref, topv_ref, yp_ref, x_ref, out_ref):
    cols = lax.broadcasted_iota(jnp.int32, (TS, P), 1)
    oh = (slots_ref[...] == cols).astype(_f32)
    y = jnp.dot(oh, yp_ref[...], preferred_element_type=_f32)
    out_ref[...] = x_ref[...] + topv_ref[...] * y


def _combine(slots_col, topv, yp, x):
    return pl.pallas_call(
        _combine_body,
        grid=(S // TS,),
        in_specs=[
            pl.BlockSpec((TS, 1), lambda i: (i, 0)),
            pl.BlockSpec((TS, 1), lambda i: (i, 0)),
            pl.BlockSpec((P, D), lambda i: (0, 0)),
            pl.BlockSpec((TS, D), lambda i: (i, 0)),
        ],
        out_specs=pl.BlockSpec((TS, D), lambda i: (i, 0)),
        out_shape=jax.ShapeDtypeStruct((S, D), _f32),
    )(slots_col, topv, yp, x)


def _route_schedule(probs, fsum, rank):
    eid = jnp.argmax(probs, axis=-1).astype(jnp.int32)
    topv = jnp.max(probs, axis=-1, keepdims=True)
    counts = fsum[0].astype(jnp.int32)
    pc = ((counts + TM - 1) // TM) * TM
    pend = jnp.cumsum(pc)
    poff = jnp.concatenate([jnp.zeros(1, jnp.int32), pend[:-1]])
    slots = jnp.take(poff, eid) + rank[:, 0].astype(jnp.int32)
    tile_e = jnp.minimum(
        jnp.searchsorted(pend, jnp.arange(NT, dtype=jnp.int32) * TM,
                         side="right"),
        E - 1).astype(jnp.int32)
    n_active = pend[-1:].astype(jnp.int32)
    return slots, topv, tile_e, n_active


def kernel(x, pos_emb, ln1_w, ln2_w, wq, wk, wv, wo, router_w, w1, w2, w3):
    xs = x.reshape(S, D) + pos_emb[:S]

    inv = 1.0 / (10000.0 ** (jnp.arange(HALF, dtype=_f32) / HALF))
    ang = jnp.arange(S, dtype=_f32)[:, None] * inv[None, :]
    cos = jnp.cos(ang)
    sin = jnp.sin(ang)

    total_aux = jnp.zeros((), _f32)
    for l in range(L):
        q, k, v = _qkv(xs, ln1_w[l], wq[l], wk[l], wv[l])
        q3 = q.reshape(S, H, DH).transpose(1, 0, 2)
        k3 = k.reshape(S, H, DH).transpose(1, 0, 2)
        v3 = v.reshape(S, H, DH).transpose(1, 0, 2)
        o = _attention(q3, k3, v3, cos, sin)
        o2 = o.transpose(1, 0, 2).reshape(S, D)
        xs = _proj_residual(o2, wo[l], xs)

        h2, probs, rank, fsum, psum = _router(xs, ln2_w[l], router_w[l])
        slots, topv, tile_e, n_active = _route_schedule(probs, fsum, rank)
        yp = _moe(h2, slots.reshape(1, S), w1[l], w3[l], w2[l],
                  tile_e, n_active)
        xs = _combine(slots.reshape(S, 1), topv, yp, xs)

        total_aux = total_aux + _f32(E) * jnp.sum(
            (fsum[0] / _f32(S)) * (psum[0] / _f32(S)))

    return xs.reshape(1, S, D), total_aux
